# baseline (speedup 1.0000x reference)
import jax
import jax.numpy as jnp
from jax.experimental import pallas as pl
from jax.experimental.pallas import tpu as pltpu


def _proj_body(x_ref, wcat_ref, bias_ref, b1_ref, b2_ref, whext_ref,
               e1_ref, d1_ref, e2_ref, d2_ref):
    xv = x_ref[...]
    whext = jnp.dot(xv, wcat_ref[...], preferred_element_type=jnp.float32)
    whext_ref[...] = whext + bias_ref[...]
    f1 = jnp.dot(xv, b1_ref[...], preferred_element_type=jnp.float32)
    f2 = jnp.dot(xv, b2_ref[...], preferred_element_type=jnp.float32)
    e1_ref[...] = jnp.exp(f1)
    d1_ref[...] = jnp.exp(0.2 * f1)
    e2_ref[...] = jnp.exp(f2)
    d2_ref[...] = jnp.exp(0.2 * f2)


def _layer1_body(nheads, hid, adj_ref, e1_ref, d1_ref, e2t_ref, d2t_ref,
                 whext_ref, w2_ref, wh2_ref):
    adjf = adj_ref[...].astype(jnp.float32)
    hp1 = hid + 1
    head_outs = []
    for h in range(nheads):
        q = jnp.maximum(e1_ref[:, h:h + 1] * e2t_ref[h:h + 1, :],
                        d1_ref[:, h:h + 1] * d2t_ref[h:h + 1, :])
        p = q * adjf
        accs = jnp.dot(p, whext_ref[:, h * hp1:(h + 1) * hp1],
                       precision=jax.lax.Precision.DEFAULT,
                       preferred_element_type=jnp.float32)
        head_outs.append(accs[:, :hid] / accs[:, hid:hid + 1])
    hb = jnp.concatenate(head_outs, axis=1)
    hb = jnp.where(hb > 0, hb, jnp.exp(hb) - 1.0)
    wh2_ref[...] = jnp.dot(hb, w2_ref[...], preferred_element_type=jnp.float32)


def _layer2_body(adj_ref, wh2_ref, wh2t_ref, ndw_ref, a2_ref, out_ref):
    adjf = adj_ref[...].astype(jnp.float32)
    wh2t = wh2t_ref[...]
    wh2 = wh2_ref[...]
    e1 = jnp.exp(wh2 * a2_ref[0:1, 0:1])
    d1 = jnp.exp(wh2 * (0.2 * a2_ref[0:1, 0:1]))
    e2 = jnp.exp(wh2t * a2_ref[0:1, 1:2])
    d2 = jnp.exp(wh2t * (0.2 * a2_ref[0:1, 1:2]))
    p = (jnp.maximum(e1 * e2, d1 * d2) * adjf).astype(jnp.bfloat16)
    nd = jnp.dot(p, ndw_ref[...], preferred_element_type=jnp.float32)
    o = nd[:, 0:1] / nd[:, 1:2]
    o = jnp.where(o > 0, o, jnp.exp(o) - 1.0)
    m = jnp.max(o, axis=1, keepdims=True)
    ls = m + jnp.log(jnp.sum(jnp.exp(o - m), axis=1, keepdims=True))
    out_ref[...] = o - ls


def kernel(x, adj, W_heads, a_heads, W2, a2):
    n, atom = x.shape
    nheads, _, hid = W_heads.shape
    feats = nheads * hid
    hp1 = hid + 1

    wcat = jnp.transpose(W_heads, (1, 0, 2)).reshape(atom, feats)
    wcat_ext = jnp.zeros((atom, nheads * hp1), jnp.float32)
    bias = jnp.zeros((1, nheads * hp1), jnp.float32)
    for h in range(nheads):
        wcat_ext = jax.lax.dynamic_update_slice(
            wcat_ext, wcat[:, h * hid:(h + 1) * hid], (0, h * hp1))
        bias = bias.at[0, h * hp1 + hid].set(1.0)
    a_src = a_heads[:, :hid, 0]
    a_dst = a_heads[:, hid:, 0]
    b1 = jnp.einsum('hdk,hk->dh', W_heads, a_src)
    b2 = jnp.einsum('hdk,hk->dh', W_heads, a_dst)

    sds = jax.ShapeDtypeStruct
    nf4 = sds((n, nheads), jnp.float32)
    whext, e1, d1, e2, d2 = pl.pallas_call(
        _proj_body,
        out_shape=(sds((n, nheads * hp1), jnp.float32), nf4, nf4, nf4, nf4),
    )(x, wcat_ext, bias, b1, b2)
    e2t = e2.T
    d2t = d2.T

    rb = next(r for r in (200, 400, 80, 40, 16, 8, 1) if n % r == 0)
    grid = (n // rb,)

    wh2 = pl.pallas_call(
        lambda *refs: _layer1_body(nheads, hid, *refs),
        grid=grid,
        in_specs=[
            pl.BlockSpec((rb, n), lambda i: (i, 0)),
            pl.BlockSpec((rb, nheads), lambda i: (i, 0)),
            pl.BlockSpec((rb, nheads), lambda i: (i, 0)),
            pl.BlockSpec((nheads, n), lambda i: (0, 0)),
            pl.BlockSpec((nheads, n), lambda i: (0, 0)),
            pl.BlockSpec((n, nheads * hp1), lambda i: (0, 0)),
            pl.BlockSpec((feats, 1), lambda i: (0, 0)),
        ],
        out_specs=pl.BlockSpec((rb, 1), lambda i: (i, 0)),
        out_shape=sds((n, 1), jnp.float32),
        compiler_params=pltpu.CompilerParams(
            dimension_semantics=("parallel",)),
    )(adj, e1, d1, e2t, d2t, whext, W2)

    wh2t = wh2.reshape(1, n)
    ndw = jnp.concatenate([wh2, jnp.ones_like(wh2)], axis=1).astype(jnp.bfloat16)
    a2r = a2.reshape(1, 2)

    out = pl.pallas_call(
        _layer2_body,
        grid=grid,
        in_specs=[
            pl.BlockSpec((rb, n), lambda i: (i, 0)),
            pl.BlockSpec((rb, 1), lambda i: (i, 0)),
            pl.BlockSpec((1, n), lambda i: (0, 0)),
            pl.BlockSpec((n, 2), lambda i: (0, 0)),
            pl.BlockSpec((1, 2), lambda i: (0, 0)),
        ],
        out_specs=pl.BlockSpec((rb, 1), lambda i: (i, 0)),
        out_shape=sds((n, 1), jnp.float32),
        compiler_params=pltpu.CompilerParams(
            dimension_semantics=("parallel",)),
    )(adj, wh2, wh2t, ndw, a2r)
    return out


# speedup vs baseline: 1.1545x; 1.1545x over previous
import jax
import jax.numpy as jnp
from jax.experimental import pallas as pl
from jax.experimental.pallas import tpu as pltpu


def _proj_body(x_ref, wcat_ref, bias_ref, b1_ref, b2_ref, whext_ref,
               e1_ref, d1_ref, e2_ref, d2_ref):
    xv = x_ref[...]
    whext = jnp.dot(xv, wcat_ref[...], preferred_element_type=jnp.float32)
    whext_ref[...] = whext + bias_ref[...]
    f1 = jnp.dot(xv, b1_ref[...], preferred_element_type=jnp.float32)
    f2 = jnp.dot(xv, b2_ref[...], preferred_element_type=jnp.float32)
    e1_ref[...] = jnp.exp(f1)
    d1_ref[...] = jnp.exp(0.2 * f1)
    e2_ref[...] = jnp.exp(f2)
    d2_ref[...] = jnp.exp(0.2 * f2)


def _layer1_body(nheads, hid, adj_ref, e1_ref, d1_ref, e2t_ref, d2t_ref,
                 whext_ref, w2_ref, wh2_ref):
    adjf = adj_ref[...].astype(jnp.float32)
    hp1 = hid + 1
    head_outs = []
    for h in range(nheads):
        q = jnp.maximum(e1_ref[:, h:h + 1] * e2t_ref[h:h + 1, :],
                        d1_ref[:, h:h + 1] * d2t_ref[h:h + 1, :])
        p = q * adjf
        accs = jnp.dot(p, whext_ref[:, h * hp1:(h + 1) * hp1],
                       precision=jax.lax.Precision.DEFAULT,
                       preferred_element_type=jnp.float32)
        head_outs.append(accs[:, :hid] / accs[:, hid:hid + 1])
    hb = jnp.concatenate(head_outs, axis=1)
    hb = jnp.where(hb > 0, hb, jnp.exp(hb) - 1.0)
    wh2_ref[...] = jnp.dot(hb, w2_ref[...], preferred_element_type=jnp.float32)


def _layer2_body(adj_ref, wh2_ref, wh2t_ref, ndw_ref, a2_ref, out_ref):
    adjf = adj_ref[...].astype(jnp.float32)
    wh2t = wh2t_ref[...]
    wh2 = wh2_ref[...]
    e1 = jnp.exp(wh2 * a2_ref[0:1, 0:1])
    d1 = jnp.exp(wh2 * (0.2 * a2_ref[0:1, 0:1]))
    e2 = jnp.exp(wh2t * a2_ref[0:1, 1:2])
    d2 = jnp.exp(wh2t * (0.2 * a2_ref[0:1, 1:2]))
    p = (jnp.maximum(e1 * e2, d1 * d2) * adjf).astype(jnp.bfloat16)
    nd = jnp.dot(p, ndw_ref[...], preferred_element_type=jnp.float32)
    o = nd[:, 0:1] / nd[:, 1:2]
    o = jnp.where(o > 0, o, jnp.exp(o) - 1.0)
    m = jnp.max(o, axis=1, keepdims=True)
    ls = m + jnp.log(jnp.sum(jnp.exp(o - m), axis=1, keepdims=True))
    out_ref[...] = o - ls


def kernel(x, adj, W_heads, a_heads, W2, a2):
    n, atom = x.shape
    nheads, _, hid = W_heads.shape
    feats = nheads * hid
    hp1 = hid + 1

    wcat = jnp.transpose(W_heads, (1, 0, 2)).reshape(atom, feats)
    wcat_ext = jnp.zeros((atom, nheads * hp1), jnp.float32)
    bias = jnp.zeros((1, nheads * hp1), jnp.float32)
    for h in range(nheads):
        wcat_ext = jax.lax.dynamic_update_slice(
            wcat_ext, wcat[:, h * hid:(h + 1) * hid], (0, h * hp1))
        bias = bias.at[0, h * hp1 + hid].set(1.0)
    a_src = a_heads[:, :hid, 0]
    a_dst = a_heads[:, hid:, 0]
    b1 = jnp.einsum('hdk,hk->dh', W_heads, a_src)
    b2 = jnp.einsum('hdk,hk->dh', W_heads, a_dst)

    sds = jax.ShapeDtypeStruct
    nf4 = sds((n, nheads), jnp.float32)
    whext, e1, d1, e2, d2 = pl.pallas_call(
        _proj_body,
        out_shape=(sds((n, nheads * hp1), jnp.float32), nf4, nf4, nf4, nf4),
    )(x, wcat_ext, bias, b1, b2)
    e2t = e2.T
    d2t = d2.T

    rb = next(r for r in (400, 200, 80, 40, 16, 8, 1) if n % r == 0)
    grid = (n // rb,)

    wh2 = pl.pallas_call(
        lambda *refs: _layer1_body(nheads, hid, *refs),
        grid=grid,
        in_specs=[
            pl.BlockSpec((rb, n), lambda i: (i, 0)),
            pl.BlockSpec((rb, nheads), lambda i: (i, 0)),
            pl.BlockSpec((rb, nheads), lambda i: (i, 0)),
            pl.BlockSpec((nheads, n), lambda i: (0, 0)),
            pl.BlockSpec((nheads, n), lambda i: (0, 0)),
            pl.BlockSpec((n, nheads * hp1), lambda i: (0, 0)),
            pl.BlockSpec((feats, 1), lambda i: (0, 0)),
        ],
        out_specs=pl.BlockSpec((rb, 1), lambda i: (i, 0)),
        out_shape=sds((n, 1), jnp.float32),
        compiler_params=pltpu.CompilerParams(
            dimension_semantics=("parallel",)),
    )(adj, e1, d1, e2t, d2t, whext, W2)

    wh2t = wh2.reshape(1, n)
    ndw = jnp.concatenate([wh2, jnp.ones_like(wh2)], axis=1).astype(jnp.bfloat16)
    a2r = a2.reshape(1, 2)

    out = pl.pallas_call(
        _layer2_body,
        grid=grid,
        in_specs=[
            pl.BlockSpec((rb, n), lambda i: (i, 0)),
            pl.BlockSpec((rb, 1), lambda i: (i, 0)),
            pl.BlockSpec((1, n), lambda i: (0, 0)),
            pl.BlockSpec((n, 2), lambda i: (0, 0)),
            pl.BlockSpec((1, 2), lambda i: (0, 0)),
        ],
        out_specs=pl.BlockSpec((rb, 1), lambda i: (i, 0)),
        out_shape=sds((n, 1), jnp.float32),
        compiler_params=pltpu.CompilerParams(
            dimension_semantics=("parallel",)),
    )(adj, wh2, wh2t, ndw, a2r)
    return out
